# baseline (speedup 1.0000x reference)
import functools

import jax
import jax.numpy as jnp
from jax import lax
from jax.experimental import pallas as pl
from jax.experimental.pallas import tpu as pltpu
from jax.experimental.pallas import tpu_sc as plsc

N = 10000
E = 320000
D = 128
L = 4
P = 5

NC = 2
NS = 16
NW = NC * NS
B = 125
C = E // (NW * B)
NB = 2
H = 4
CH = C // H
RPS = 624
TAIL0 = NS * RPS
TAIL = N - TAIL0


def _sc_segment_sum(x, src3, dst3):
  mesh = plsc.VectorSubcoreMesh(core_axis_name="c", subcore_axis_name="s")

  @functools.partial(
      pl.kernel,
      mesh=mesh,
      out_type=jax.ShapeDtypeStruct((NC, N, D), jnp.float32),
      scratch_types=[
          pltpu.VMEM((CH, B), jnp.int32),
          pltpu.VMEM((CH, B), jnp.int32),
          *[pltpu.VMEM((B, D), jnp.float32) for _ in range(NB)],
          pltpu.VMEM_SHARED((N, D), jnp.float32),
          *[pltpu.SemaphoreType.DMA for _ in range(NB)],
      ],
  )
  def k(x_hbm, src_hbm, dst_hbm, out_hbm, src_v, dst_v, *rest):
    rows = rest[:NB]
    agg_sh = rest[NB]
    gsem = rest[NB + 1:NB + 1 + NB]
    cid = lax.axis_index("c")
    sid = lax.axis_index("s")
    wid = sid * NC + cid

    row0 = pl.multiple_of(sid * RPS, 8)
    pltpu.sync_copy(x_hbm.at[pl.ds(row0, RPS)], agg_sh.at[pl.ds(row0, RPS)])

    @pl.when(sid == NS - 1)
    def _():
      pltpu.sync_copy(x_hbm.at[pl.ds(TAIL0, TAIL)],
                      agg_sh.at[pl.ds(TAIL0, TAIL)])

    plsc.subcore_barrier()

    @pl.loop(0, H)
    def _(h):
      pltpu.sync_copy(src_hbm.at[wid, h], src_v)
      pltpu.sync_copy(dst_hbm.at[wid, h], dst_v)

      rows0, rows1 = rows
      sem0, sem1 = gsem
      pltpu.async_copy(x_hbm.at[src_v.at[0]], rows0, sem0)
      pltpu.async_copy(x_hbm.at[src_v.at[1]], rows1, sem1)

      @pl.loop(0, CH - 2, step=2)
      def _(c):
        pltpu.make_async_copy(x_hbm.at[src_v.at[c]], rows0, sem0).wait()
        pltpu.sync_copy(rows0, agg_sh.at[dst_v.at[c]], add=True)
        pltpu.async_copy(x_hbm.at[src_v.at[c + 2]], rows0, sem0)
        pltpu.make_async_copy(x_hbm.at[src_v.at[c + 1]], rows1, sem1).wait()
        pltpu.sync_copy(rows1, agg_sh.at[dst_v.at[c + 1]], add=True)
        pltpu.async_copy(x_hbm.at[src_v.at[c + 3]], rows1, sem1)

      pltpu.make_async_copy(x_hbm.at[src_v.at[CH - 2]], rows0, sem0).wait()
      pltpu.sync_copy(rows0, agg_sh.at[dst_v.at[CH - 2]], add=True)
      pltpu.make_async_copy(x_hbm.at[src_v.at[CH - 1]], rows1, sem1).wait()
      pltpu.sync_copy(rows1, agg_sh.at[dst_v.at[CH - 1]], add=True)

    plsc.subcore_barrier()
    pltpu.sync_copy(agg_sh.at[pl.ds(row0, RPS)],
                    out_hbm.at[cid, pl.ds(row0, RPS)])

    @pl.when(sid == NS - 1)
    def _():
      pltpu.sync_copy(agg_sh.at[pl.ds(TAIL0, TAIL)],
                      out_hbm.at[cid, pl.ds(TAIL0, TAIL)])

  return k(x, src3, dst3)


def _dot_t(a, w):
  return lax.dot_general(a, w, (((1,), (1,)), ((), ())),
                         preferred_element_type=jnp.float32,
                         precision=lax.Precision.HIGHEST)


def _bn_relu(z, g, b):
  mu = jnp.mean(z, axis=0, keepdims=True)
  ex2 = jnp.mean(z * z, axis=0, keepdims=True)
  var = ex2 - mu * mu
  scale = g * lax.rsqrt(var + 1e-5)
  return jnp.maximum(z * scale + (b - mu * scale), 0.0)


def _tc_layer_body(last, x_ref, agg_ref, w1_ref, w2_ref, g1_ref, b1_ref,
                   g2_ref, b2_ref, pw_ref, pb_ref, score_ref, out_ref,
                   score_out_ref):
  x = x_ref[...]
  pooled = jnp.sum(x, axis=0, keepdims=True)
  score = score_ref[...] + _dot_t(pooled, pw_ref[0])
  s = agg_ref[0] + agg_ref[1] - x
  z = _dot_t(s, w1_ref[...])
  z = _bn_relu(z, g1_ref[...], b1_ref[...])
  z = _dot_t(z, w2_ref[...])
  out = _bn_relu(z, g2_ref[...], b2_ref[...])
  out_ref[...] = out
  if last:
    pooled = jnp.sum(out, axis=0, keepdims=True)
    score = score + _dot_t(pooled, pw_ref[1])
    score = score + jnp.sum(pb_ref[...], axis=0, keepdims=True)
  score_out_ref[...] = score


def _tc_layer(last, x, agg, w1, w2, g1, b1, g2, b2, pw, pb, score):
  return pl.pallas_call(
      functools.partial(_tc_layer_body, last),
      out_shape=(jax.ShapeDtypeStruct((N, D), jnp.float32),
                 jax.ShapeDtypeStruct((1, D), jnp.float32)),
  )(x, agg, w1, w2, g1.reshape(1, D), b1.reshape(1, D),
    g2.reshape(1, D), b2.reshape(1, D), pw, pb, score)


def kernel(h, edge_index, mlp_W1, mlp_W2, mlp_bn_g, mlp_bn_b, bn_g, bn_b,
           pred_W, pred_b):
  src3 = edge_index[0].reshape(NW, H, CH, B)
  dst3 = edge_index[1].reshape(NW, H, CH, B)
  x = h
  score = jnp.zeros((1, D), dtype=jnp.float32)
  for i in range(L):
    agg = _sc_segment_sum(x, src3, dst3)
    last = i == L - 1
    pw = pred_W[i:i + 2] if last else pred_W[i:i + 1]
    x, score = _tc_layer(last, x, agg, mlp_W1[i], mlp_W2[i], mlp_bn_g[i],
                         mlp_bn_b[i], bn_g[i], bn_b[i], pw, pred_b, score)
  return score


# speedup vs baseline: 1.0299x; 1.0299x over previous
import functools

import jax
import jax.numpy as jnp
from jax import lax
from jax.experimental import pallas as pl
from jax.experimental.pallas import tpu as pltpu
from jax.experimental.pallas import tpu_sc as plsc

N = 10000
E = 320000
D = 128
L = 4
P = 5

NC = 2
NS = 16
NW = NC * NS
B = 100
C = E // (NW * B)
NB = 2
H = 2
CH = C // H
RPS = 624
TAIL0 = NS * RPS
TAIL = N - TAIL0


def _sc_segment_sum(x, src3, dst3):
  mesh = plsc.VectorSubcoreMesh(core_axis_name="c", subcore_axis_name="s")

  @functools.partial(
      pl.kernel,
      mesh=mesh,
      out_type=jax.ShapeDtypeStruct((NC, N, D), jnp.float32),
      scratch_types=[
          pltpu.VMEM((CH, B), jnp.int32),
          pltpu.VMEM((CH, B), jnp.int32),
          *[pltpu.VMEM((B, D), jnp.float32) for _ in range(NB)],
          pltpu.VMEM_SHARED((N, D), jnp.float32),
          *[pltpu.SemaphoreType.DMA for _ in range(NB + 1)],
      ],
  )
  def k(x_hbm, src_hbm, dst_hbm, out_hbm, src_v, dst_v, *rest):
    rows = rest[:NB]
    agg_sh = rest[NB]
    gsem = rest[NB + 1:NB + 1 + NB]
    isem = rest[NB + 1 + NB]
    cid = lax.axis_index("c")
    sid = lax.axis_index("s")
    wid = sid * NC + cid

    row0 = pl.multiple_of(sid * RPS, 8)
    pltpu.async_copy(x_hbm.at[pl.ds(row0, RPS)], agg_sh.at[pl.ds(row0, RPS)],
                     isem)

    @pl.when(sid == NS - 1)
    def _():
      pltpu.async_copy(x_hbm.at[pl.ds(TAIL0, TAIL)],
                       agg_sh.at[pl.ds(TAIL0, TAIL)], isem)

    rows0, rows1 = rows
    sem0, sem1 = gsem
    for h in range(H):
      pltpu.sync_copy(src_hbm.at[wid, h], src_v)
      pltpu.sync_copy(dst_hbm.at[wid, h], dst_v)

      pltpu.async_copy(x_hbm.at[src_v.at[0]], rows0, sem0)
      pltpu.async_copy(x_hbm.at[src_v.at[1]], rows1, sem1)

      if h == 0:
        pltpu.make_async_copy(x_hbm.at[pl.ds(row0, RPS)],
                              agg_sh.at[pl.ds(row0, RPS)], isem).wait()

        @pl.when(sid == NS - 1)
        def _():
          pltpu.make_async_copy(x_hbm.at[pl.ds(TAIL0, TAIL)],
                                agg_sh.at[pl.ds(TAIL0, TAIL)], isem).wait()

        plsc.subcore_barrier()

      @pl.loop(0, CH - 2, step=2)
      def _(c):
        pltpu.make_async_copy(x_hbm.at[src_v.at[c]], rows0, sem0).wait()
        pltpu.sync_copy(rows0, agg_sh.at[dst_v.at[c]], add=True)
        pltpu.async_copy(x_hbm.at[src_v.at[c + 2]], rows0, sem0)
        pltpu.make_async_copy(x_hbm.at[src_v.at[c + 1]], rows1, sem1).wait()
        pltpu.sync_copy(rows1, agg_sh.at[dst_v.at[c + 1]], add=True)
        pltpu.async_copy(x_hbm.at[src_v.at[c + 3]], rows1, sem1)

      pltpu.make_async_copy(x_hbm.at[src_v.at[CH - 2]], rows0, sem0).wait()
      pltpu.sync_copy(rows0, agg_sh.at[dst_v.at[CH - 2]], add=True)
      pltpu.make_async_copy(x_hbm.at[src_v.at[CH - 1]], rows1, sem1).wait()
      pltpu.sync_copy(rows1, agg_sh.at[dst_v.at[CH - 1]], add=True)

    plsc.subcore_barrier()
    pltpu.sync_copy(agg_sh.at[pl.ds(row0, RPS)],
                    out_hbm.at[cid, pl.ds(row0, RPS)])

    @pl.when(sid == NS - 1)
    def _():
      pltpu.sync_copy(agg_sh.at[pl.ds(TAIL0, TAIL)],
                      out_hbm.at[cid, pl.ds(TAIL0, TAIL)])

  return k(x, src3, dst3)


def _dot_t(a, w):
  return lax.dot_general(a, w, (((1,), (1,)), ((), ())),
                         preferred_element_type=jnp.float32,
                         precision=lax.Precision.HIGHEST)


def _bn_relu(z, g, b):
  mu = jnp.mean(z, axis=0, keepdims=True)
  ex2 = jnp.mean(z * z, axis=0, keepdims=True)
  var = ex2 - mu * mu
  scale = g * lax.rsqrt(var + 1e-5)
  return jnp.maximum(z * scale + (b - mu * scale), 0.0)


def _tc_layer_body(last, x_ref, agg_ref, w1_ref, w2_ref, g1_ref, b1_ref,
                   g2_ref, b2_ref, pw_ref, pb_ref, score_ref, out_ref,
                   score_out_ref):
  x = x_ref[...]
  pooled = jnp.sum(x, axis=0, keepdims=True)
  score = score_ref[...] + _dot_t(pooled, pw_ref[0])
  s = agg_ref[0] + agg_ref[1] - x
  z = _dot_t(s, w1_ref[...])
  z = _bn_relu(z, g1_ref[...], b1_ref[...])
  z = _dot_t(z, w2_ref[...])
  out = _bn_relu(z, g2_ref[...], b2_ref[...])
  out_ref[...] = out
  if last:
    pooled = jnp.sum(out, axis=0, keepdims=True)
    score = score + _dot_t(pooled, pw_ref[1])
    score = score + jnp.sum(pb_ref[...], axis=0, keepdims=True)
  score_out_ref[...] = score


def _tc_layer(last, x, agg, w1, w2, g1, b1, g2, b2, pw, pb, score):
  return pl.pallas_call(
      functools.partial(_tc_layer_body, last),
      out_shape=(jax.ShapeDtypeStruct((N, D), jnp.float32),
                 jax.ShapeDtypeStruct((1, D), jnp.float32)),
  )(x, agg, w1, w2, g1.reshape(1, D), b1.reshape(1, D),
    g2.reshape(1, D), b2.reshape(1, D), pw, pb, score)


def kernel(h, edge_index, mlp_W1, mlp_W2, mlp_bn_g, mlp_bn_b, bn_g, bn_b,
           pred_W, pred_b):
  src3 = edge_index[0].reshape(NW, H, CH, B)
  dst3 = edge_index[1].reshape(NW, H, CH, B)
  x = h
  score = jnp.zeros((1, D), dtype=jnp.float32)
  for i in range(L):
    agg = _sc_segment_sum(x, src3, dst3)
    last = i == L - 1
    pw = pred_W[i:i + 2] if last else pred_W[i:i + 1]
    x, score = _tc_layer(last, x, agg, mlp_W1[i], mlp_W2[i], mlp_bn_g[i],
                         mlp_bn_b[i], bn_g[i], bn_b[i], pw, pred_b, score)
  return score


# speedup vs baseline: 1.1759x; 1.1417x over previous
import functools

import jax
import jax.numpy as jnp
from jax import lax
from jax.experimental import pallas as pl
from jax.experimental.pallas import tpu as pltpu
from jax.experimental.pallas import tpu_sc as plsc

N = 10000
E = 320000
D = 128
L = 4
P = 5

NC = 2
NS = 16
NW = NC * NS
B = 100
C = E // (NW * B)
NB = 2
H = 2
CH = C // H
RPS = 624
TAIL0 = NS * RPS
TAIL = N - TAIL0


def _sc_segment_sum(x, src3, dst3):
  mesh = plsc.VectorSubcoreMesh(core_axis_name="c", subcore_axis_name="s")

  @functools.partial(
      pl.kernel,
      mesh=mesh,
      out_type=jax.ShapeDtypeStruct((NC, N, D), jnp.float32),
      scratch_types=[
          pltpu.VMEM((CH, B), jnp.int32),
          pltpu.VMEM((CH, B), jnp.int32),
          *[pltpu.VMEM((B, D), jnp.float32) for _ in range(NB)],
          pltpu.VMEM_SHARED((N, D), jnp.float32),
          *[pltpu.SemaphoreType.DMA for _ in range(NB + 1)],
      ],
  )
  def k(x_hbm, src_hbm, dst_hbm, out_hbm, src_v, dst_v, *rest):
    rows = rest[:NB]
    agg_sh = rest[NB]
    gsem = rest[NB + 1:NB + 1 + NB]
    isem = rest[NB + 1 + NB]
    cid = lax.axis_index("c")
    sid = lax.axis_index("s")
    wid = sid * NC + cid

    row0 = pl.multiple_of(sid * RPS, 8)
    pltpu.async_copy(x_hbm.at[pl.ds(row0, RPS)], agg_sh.at[pl.ds(row0, RPS)],
                     isem)

    @pl.when(sid == NS - 1)
    def _():
      pltpu.async_copy(x_hbm.at[pl.ds(TAIL0, TAIL)],
                       agg_sh.at[pl.ds(TAIL0, TAIL)], isem)

    rows0, rows1 = rows
    sem0, sem1 = gsem
    for h in range(H):
      pltpu.sync_copy(src_hbm.at[wid, h], src_v)
      pltpu.sync_copy(dst_hbm.at[wid, h], dst_v)

      pltpu.async_copy(x_hbm.at[src_v.at[0]], rows0, sem0)
      pltpu.async_copy(x_hbm.at[src_v.at[1]], rows1, sem1)

      if h == 0:
        pltpu.make_async_copy(x_hbm.at[pl.ds(row0, RPS)],
                              agg_sh.at[pl.ds(row0, RPS)], isem).wait()

        @pl.when(sid == NS - 1)
        def _():
          pltpu.make_async_copy(x_hbm.at[pl.ds(TAIL0, TAIL)],
                                agg_sh.at[pl.ds(TAIL0, TAIL)], isem).wait()

        plsc.subcore_barrier()

      @pl.loop(0, CH - 2, step=2)
      def _(c):
        pltpu.make_async_copy(x_hbm.at[src_v.at[c]], rows0, sem0).wait()
        pltpu.sync_copy(rows0, agg_sh.at[dst_v.at[c]], add=True)
        pltpu.async_copy(x_hbm.at[src_v.at[c + 2]], rows0, sem0)
        pltpu.make_async_copy(x_hbm.at[src_v.at[c + 1]], rows1, sem1).wait()
        pltpu.sync_copy(rows1, agg_sh.at[dst_v.at[c + 1]], add=True)
        pltpu.async_copy(x_hbm.at[src_v.at[c + 3]], rows1, sem1)

      pltpu.make_async_copy(x_hbm.at[src_v.at[CH - 2]], rows0, sem0).wait()
      pltpu.sync_copy(rows0, agg_sh.at[dst_v.at[CH - 2]], add=True)
      pltpu.make_async_copy(x_hbm.at[src_v.at[CH - 1]], rows1, sem1).wait()
      pltpu.sync_copy(rows1, agg_sh.at[dst_v.at[CH - 1]], add=True)

    plsc.subcore_barrier()
    pltpu.sync_copy(agg_sh.at[pl.ds(row0, RPS)],
                    out_hbm.at[cid, pl.ds(row0, RPS)])

    @pl.when(sid == NS - 1)
    def _():
      pltpu.sync_copy(agg_sh.at[pl.ds(TAIL0, TAIL)],
                      out_hbm.at[cid, pl.ds(TAIL0, TAIL)])

  return k(x, src3, dst3)


def _dot_t(a, w):
  return lax.dot_general(a, w, (((1,), (1,)), ((), ())),
                         preferred_element_type=jnp.float32,
                         precision=lax.Precision.DEFAULT)


def _bn_relu(z, g, b):
  mu = jnp.mean(z, axis=0, keepdims=True)
  ex2 = jnp.mean(z * z, axis=0, keepdims=True)
  var = ex2 - mu * mu
  scale = g * lax.rsqrt(var + 1e-5)
  return jnp.maximum(z * scale + (b - mu * scale), 0.0)


def _tc_layer_body(last, x_ref, agg_ref, w1_ref, w2_ref, g1_ref, b1_ref,
                   g2_ref, b2_ref, pw_ref, pb_ref, score_ref, out_ref,
                   score_out_ref):
  x = x_ref[...]
  pooled = jnp.sum(x, axis=0, keepdims=True)
  score = score_ref[...] + _dot_t(pooled, pw_ref[0])
  s = agg_ref[0] + agg_ref[1] - x
  z = _dot_t(s, w1_ref[...])
  z = _bn_relu(z, g1_ref[...], b1_ref[...])
  z = _dot_t(z, w2_ref[...])
  out = _bn_relu(z, g2_ref[...], b2_ref[...])
  out_ref[...] = out
  if last:
    pooled = jnp.sum(out, axis=0, keepdims=True)
    score = score + _dot_t(pooled, pw_ref[1])
    score = score + jnp.sum(pb_ref[...], axis=0, keepdims=True)
  score_out_ref[...] = score


def _tc_layer(last, x, agg, w1, w2, g1, b1, g2, b2, pw, pb, score):
  return pl.pallas_call(
      functools.partial(_tc_layer_body, last),
      out_shape=(jax.ShapeDtypeStruct((N, D), jnp.float32),
                 jax.ShapeDtypeStruct((1, D), jnp.float32)),
  )(x, agg, w1, w2, g1.reshape(1, D), b1.reshape(1, D),
    g2.reshape(1, D), b2.reshape(1, D), pw, pb, score)


def kernel(h, edge_index, mlp_W1, mlp_W2, mlp_bn_g, mlp_bn_b, bn_g, bn_b,
           pred_W, pred_b):
  src3 = edge_index[0].reshape(NW, H, CH, B)
  dst3 = edge_index[1].reshape(NW, H, CH, B)
  x = h
  score = jnp.zeros((1, D), dtype=jnp.float32)
  for i in range(L):
    agg = _sc_segment_sum(x, src3, dst3)
    last = i == L - 1
    pw = pred_W[i:i + 2] if last else pred_W[i:i + 1]
    x, score = _tc_layer(last, x, agg, mlp_W1[i], mlp_W2[i], mlp_bn_g[i],
                         mlp_bn_b[i], bn_g[i], bn_b[i], pw, pred_b, score)
  return score
